# baseline (speedup 1.0000x reference)
import functools

import jax
import jax.numpy as jnp
from jax import lax
from jax.experimental import pallas as pl
from jax.experimental.pallas import tpu as pltpu
from jax.experimental.pallas import tpu_sc as plsc


def kernel(arr):
    B, S2, D = arr.shape
    S = int(round(S2 ** 0.5))
    assert S * S == S2
    out_rows = S2 - S
    blk = S * D
    src_stride = (S + 1) * D

    mesh = plsc.VectorSubcoreMesh(core_axis_name="c", subcore_axis_name="s")

    @functools.partial(
        pl.kernel,
        mesh=mesh,
        out_type=jax.ShapeDtypeStruct((B * out_rows * D,), arr.dtype),
        scratch_types=[
            pltpu.VMEM((blk,), arr.dtype),
            pltpu.SemaphoreType.DMA,
        ],
    )
    def copy_offdiag(arr_hbm, out_hbm, buf, sem):
        wid = lax.axis_index("s") * 2 + lax.axis_index("c")
        src_base = wid * (S2 * D)
        dst_base = wid * (out_rows * D)

        def body(d, carry):
            src = arr_hbm.at[pl.ds(src_base + src_stride * d + D, blk)]
            dst = out_hbm.at[pl.ds(dst_base + blk * d, blk)]
            pltpu.async_copy(src, buf, sem).wait()
            pltpu.async_copy(buf, dst, sem).wait()
            return carry

        lax.fori_loop(0, S - 1, body, None)

    out_flat = copy_offdiag(arr.reshape(-1))
    return out_flat.reshape(B, out_rows, D)


# speedup vs baseline: 1.1293x; 1.1293x over previous
import functools

import jax
import jax.numpy as jnp
from jax import lax
from jax.experimental import pallas as pl
from jax.experimental.pallas import tpu as pltpu
from jax.experimental.pallas import tpu_sc as plsc

_NBUF = 8
_LAG = 4


def kernel(arr):
    B, S2, D = arr.shape
    S = int(round(S2 ** 0.5))
    assert S * S == S2
    out_rows = S2 - S
    nblk = S - 1
    blk = S * D
    src_stride = (S + 1) * D

    mesh = plsc.VectorSubcoreMesh(core_axis_name="c", subcore_axis_name="s")

    @functools.partial(
        pl.kernel,
        mesh=mesh,
        out_type=jax.ShapeDtypeStruct((B * out_rows * D,), arr.dtype),
        scratch_types=(
            [pltpu.VMEM((_NBUF * blk,), arr.dtype)]
            + [pltpu.SemaphoreType.DMA] * (2 * _NBUF)
        ),
    )
    def copy_offdiag(arr_hbm, out_hbm, buf, *sems):
        sem_g = sems[:_NBUF]
        sem_s = sems[_NBUF:]
        wid = lax.axis_index("s") * 2 + lax.axis_index("c")
        src_base = wid * (S2 * D)
        dst_base = wid * (out_rows * D)

        def gather(d):
            p = d % _NBUF
            src = arr_hbm.at[pl.ds(src_base + src_stride * d + D, blk)]
            if d >= _NBUF:
                pltpu.make_async_copy(
                    buf.at[pl.ds(p * blk, blk)],
                    out_hbm.at[pl.ds(dst_base + blk * (d - _NBUF), blk)],
                    sem_s[p],
                ).wait()
            pltpu.async_copy(src, buf.at[pl.ds(p * blk, blk)], sem_g[p])

        def scatter(d):
            p = d % _NBUF
            src = arr_hbm.at[pl.ds(src_base + src_stride * d + D, blk)]
            pltpu.make_async_copy(src, buf.at[pl.ds(p * blk, blk)],
                                  sem_g[p]).wait()
            pltpu.async_copy(buf.at[pl.ds(p * blk, blk)],
                             out_hbm.at[pl.ds(dst_base + blk * d, blk)],
                             sem_s[p])

        for d in range(nblk + _LAG):
            if d < nblk:
                gather(d)
            if d >= _LAG:
                scatter(d - _LAG)

        for d in range(max(0, nblk - _NBUF), nblk):
            p = d % _NBUF
            pltpu.make_async_copy(
                buf.at[pl.ds(p * blk, blk)],
                out_hbm.at[pl.ds(dst_base + blk * d, blk)],
                sem_s[p],
            ).wait()

    out_flat = copy_offdiag(arr.reshape(-1))
    return out_flat.reshape(B, out_rows, D)


# speedup vs baseline: 1.1302x; 1.0008x over previous
import functools

import jax
import jax.numpy as jnp
from jax import lax
from jax.experimental import pallas as pl
from jax.experimental.pallas import tpu as pltpu
from jax.experimental.pallas import tpu_sc as plsc

_G = 7


def kernel(arr):
    B, S2, D = arr.shape
    S = int(round(S2 ** 0.5))
    assert S * S == S2
    out_rows = S2 - S
    nblk = S - 1
    blk = S * D
    stride = (S + 1) * D

    chunks = [(k * _G, _G) for k in range(nblk // _G)]
    if nblk % _G:
        chunks.append((nblk - nblk % _G, nblk % _G))
    span = stride * (_G - 1) + blk
    slot = span

    mesh = plsc.VectorSubcoreMesh(core_axis_name="c", subcore_axis_name="s")

    @functools.partial(
        pl.kernel,
        mesh=mesh,
        out_type=jax.ShapeDtypeStruct((B * out_rows * D,), arr.dtype),
        scratch_types=(
            [pltpu.VMEM((2 * slot,), arr.dtype)]
            + [pltpu.SemaphoreType.DMA] * 4
        ),
    )
    def copy_offdiag(arr_hbm, out_hbm, buf, sg0, sg1, ss0, ss1):
        sem_g = (sg0, sg1)
        sem_s = (ss0, ss1)
        wid = lax.axis_index("s") * 2 + lax.axis_index("c")
        src_base = wid * (S2 * D)
        dst_base = wid * (out_rows * D)

        def span_len(g):
            return stride * (g - 1) + blk

        def gather(i):
            d0, g = chunks[i]
            p = i % 2
            pltpu.async_copy(
                arr_hbm.at[pl.ds(src_base + stride * d0 + D, span_len(g))],
                buf.at[pl.ds(p * slot, span_len(g))],
                sem_g[p],
            )

        def wait_gather(i):
            d0, g = chunks[i]
            p = i % 2
            pltpu.make_async_copy(
                arr_hbm.at[pl.ds(src_base + stride * d0 + D, span_len(g))],
                buf.at[pl.ds(p * slot, span_len(g))],
                sem_g[p],
            ).wait()

        def scatter(i):
            d0, g = chunks[i]
            p = i % 2
            for j in range(g):
                pltpu.async_copy(
                    buf.at[pl.ds(p * slot + stride * j, blk)],
                    out_hbm.at[pl.ds(dst_base + blk * (d0 + j), blk)],
                    sem_s[p],
                )

        def wait_scatter(i):
            d0, g = chunks[i]
            p = i % 2
            pltpu.make_async_copy(
                buf.at[pl.ds(p * slot, blk * g)],
                out_hbm.at[pl.ds(dst_base + blk * d0, blk * g)],
                sem_s[p],
            ).wait()

        n = len(chunks)
        for i in range(n + 1):
            if i < n:
                if i >= 2:
                    wait_scatter(i - 2)
                gather(i)
            if i >= 1:
                wait_gather(i - 1)
                scatter(i - 1)
        for i in range(max(0, n - 2), n):
            wait_scatter(i)

    out_flat = copy_offdiag(arr.reshape(-1))
    return out_flat.reshape(B, out_rows, D)


# speedup vs baseline: 1.1347x; 1.0040x over previous
import functools

import jax
import jax.numpy as jnp
from jax import lax
from jax.experimental import pallas as pl
from jax.experimental.pallas import tpu as pltpu
from jax.experimental.pallas import tpu_sc as plsc

_G = 7


def kernel(arr):
    B, S2, D = arr.shape
    S = int(round(S2 ** 0.5))
    assert S * S == S2
    out_rows = S2 - S
    nblk = S - 1
    blk = S * D
    stride = (S + 1) * D

    chunks = [(k * _G, _G) for k in range(nblk // _G)]
    if nblk % _G:
        chunks.append((nblk - nblk % _G, nblk % _G))
    slot = blk * _G

    mesh = plsc.VectorSubcoreMesh(core_axis_name="c", subcore_axis_name="s")

    @functools.partial(
        pl.kernel,
        mesh=mesh,
        out_type=jax.ShapeDtypeStruct((B * out_rows * D,), arr.dtype),
        scratch_types=(
            [pltpu.VMEM((2 * slot,), arr.dtype)]
            + [pltpu.SemaphoreType.DMA] * 4
        ),
    )
    def copy_offdiag(arr_hbm, out_hbm, buf, sg0, sg1, ss0, ss1):
        sem_g = (sg0, sg1)
        sem_s = (ss0, ss1)
        wid = lax.axis_index("s") * 2 + lax.axis_index("c")
        src_base = wid * (S2 * D)
        dst_base = wid * (out_rows * D)

        def gather(i):
            d0, g = chunks[i]
            p = i % 2
            for j in range(g):
                pltpu.async_copy(
                    arr_hbm.at[pl.ds(src_base + stride * (d0 + j) + D, blk)],
                    buf.at[pl.ds(p * slot + blk * j, blk)],
                    sem_g[p],
                )

        def wait_gather(i):
            d0, g = chunks[i]
            p = i % 2
            pltpu.make_async_copy(
                arr_hbm.at[pl.ds(src_base, blk * g)],
                buf.at[pl.ds(p * slot, blk * g)],
                sem_g[p],
            ).wait()

        def scatter(i):
            d0, g = chunks[i]
            p = i % 2
            pltpu.async_copy(
                buf.at[pl.ds(p * slot, blk * g)],
                out_hbm.at[pl.ds(dst_base + blk * d0, blk * g)],
                sem_s[p],
            )

        def wait_scatter(i):
            d0, g = chunks[i]
            p = i % 2
            pltpu.make_async_copy(
                buf.at[pl.ds(p * slot, blk * g)],
                out_hbm.at[pl.ds(dst_base + blk * d0, blk * g)],
                sem_s[p],
            ).wait()

        n = len(chunks)
        for i in range(n + 1):
            if i < n:
                if i >= 2:
                    wait_scatter(i - 2)
                gather(i)
            if i >= 1:
                wait_gather(i - 1)
                scatter(i - 1)
        for i in range(max(0, n - 2), n):
            wait_scatter(i)

    out_flat = copy_offdiag(arr.reshape(-1))
    return out_flat.reshape(B, out_rows, D)


# speedup vs baseline: 1.4023x; 1.2358x over previous
import functools

import jax
import jax.numpy as jnp
from jax import lax
from jax.experimental import pallas as pl
from jax.experimental.pallas import tpu as pltpu
from jax.experimental.pallas import tpu_sc as plsc

_NBUF = 4
_LAG = 2


def kernel(arr):
    B, S2, D = arr.shape
    S = int(round(S2 ** 0.5))
    assert S * S == S2
    out_rows = S2 - S
    nblk = S - 1
    brows = S + 8

    mesh = plsc.VectorSubcoreMesh(core_axis_name="c", subcore_axis_name="s")

    @functools.partial(
        pl.kernel,
        mesh=mesh,
        out_type=jax.ShapeDtypeStruct((B, out_rows, D), arr.dtype),
        scratch_types=(
            [pltpu.VMEM((_NBUF * brows, D), arr.dtype)]
            + [pltpu.SemaphoreType.DMA] * (2 * _NBUF)
        ),
    )
    def copy_offdiag(arr_hbm, out_hbm, buf, *sems):
        sem_g = sems[:_NBUF]
        sem_s = sems[_NBUF:]
        wid = lax.axis_index("s") * 2 + lax.axis_index("c")
        b = wid

        def src_geom(d):
            r0 = (S + 1) * d + 1
            ph = r0 % 8
            nr = S + 8 if ph else S
            return r0 - ph, ph, nr

        def gather(d):
            p = d % _NBUF
            a0, ph, nr = src_geom(d)
            pltpu.async_copy(
                arr_hbm.at[b, pl.ds(a0, nr), :],
                buf.at[pl.ds(p * brows, nr), :],
                sem_g[p],
            )

        def wait_gather(d):
            p = d % _NBUF
            a0, ph, nr = src_geom(d)
            pltpu.make_async_copy(
                arr_hbm.at[b, pl.ds(a0, nr), :],
                buf.at[pl.ds(p * brows, nr), :],
                sem_g[p],
            ).wait()

        def scatter(d):
            p = d % _NBUF
            a0, ph, nr = src_geom(d)
            pltpu.async_copy(
                buf.at[pl.ds(p * brows + ph, S), :],
                out_hbm.at[b, pl.ds(S * d, S), :],
                sem_s[p],
            )

        def wait_scatter(d):
            p = d % _NBUF
            a0, ph, nr = src_geom(d)
            pltpu.make_async_copy(
                buf.at[pl.ds(p * brows + ph, S), :],
                out_hbm.at[b, pl.ds(S * d, S), :],
                sem_s[p],
            ).wait()

        for d in range(nblk + _LAG):
            if d < nblk:
                if d >= _NBUF:
                    wait_scatter(d - _NBUF)
                gather(d)
            if d >= _LAG:
                wait_gather(d - _LAG)
                scatter(d - _LAG)
        for d in range(max(0, nblk - _NBUF), nblk):
            wait_scatter(d)

    return copy_offdiag(arr)


# speedup vs baseline: 1.4066x; 1.0031x over previous
import functools

import jax
import jax.numpy as jnp
from jax import lax
from jax.experimental import pallas as pl
from jax.experimental.pallas import tpu as pltpu
from jax.experimental.pallas import tpu_sc as plsc

_NBUF = 4
_LAG = 2


def kernel(arr):
    B, S2, D = arr.shape
    S = int(round(S2 ** 0.5))
    assert S * S == S2
    out_rows = S2 - S
    nblk = S - 1
    brows = S + 8

    mesh = plsc.VectorSubcoreMesh(core_axis_name="c", subcore_axis_name="s")

    @functools.partial(
        pl.kernel,
        mesh=mesh,
        out_type=jax.ShapeDtypeStruct((B, out_rows, D), arr.dtype),
        scratch_types=(
            [pltpu.VMEM((_NBUF * brows, D), arr.dtype)]
            + [pltpu.SemaphoreType.DMA] * (2 * _NBUF)
        ),
        compiler_params=pltpu.CompilerParams(use_tc_tiling_on_sc=True),
    )
    def copy_offdiag(arr_hbm, out_hbm, buf, *sems):
        sem_g = sems[:_NBUF]
        sem_s = sems[_NBUF:]
        wid = lax.axis_index("s") * 2 + lax.axis_index("c")
        b = wid

        def src_geom(d):
            r0 = (S + 1) * d + 1
            ph = r0 % 8
            nr = S + 8 if ph else S
            return r0 - ph, ph, nr

        def gather(d):
            p = d % _NBUF
            a0, ph, nr = src_geom(d)
            pltpu.async_copy(
                arr_hbm.at[b, pl.ds(a0, nr), :],
                buf.at[pl.ds(p * brows, nr), :],
                sem_g[p],
            )

        def wait_gather(d):
            p = d % _NBUF
            a0, ph, nr = src_geom(d)
            pltpu.make_async_copy(
                arr_hbm.at[b, pl.ds(a0, nr), :],
                buf.at[pl.ds(p * brows, nr), :],
                sem_g[p],
            ).wait()

        def scatter(d):
            p = d % _NBUF
            a0, ph, nr = src_geom(d)
            pltpu.async_copy(
                buf.at[pl.ds(p * brows + ph, S), :],
                out_hbm.at[b, pl.ds(S * d, S), :],
                sem_s[p],
            )

        def wait_scatter(d):
            p = d % _NBUF
            a0, ph, nr = src_geom(d)
            pltpu.make_async_copy(
                buf.at[pl.ds(p * brows + ph, S), :],
                out_hbm.at[b, pl.ds(S * d, S), :],
                sem_s[p],
            ).wait()

        for d in range(nblk + _LAG):
            if d < nblk:
                if d >= _NBUF:
                    wait_scatter(d - _NBUF)
                gather(d)
            if d >= _LAG:
                wait_gather(d - _LAG)
                scatter(d - _LAG)
        for d in range(max(0, nblk - _NBUF), nblk):
            wait_scatter(d)

    return copy_offdiag(arr)


# speedup vs baseline: 2.4907x; 1.7708x over previous
import functools

import jax
import jax.numpy as jnp
from jax import lax
from jax.experimental import pallas as pl
from jax.experimental.pallas import tpu as pltpu
from jax.experimental.pallas import tpu_sc as plsc

_G = 4
_NBUF = 3


def kernel(arr):
    B, S2, D = arr.shape
    S = int(round(S2 ** 0.5))
    assert S * S == S2
    out_rows = S2 - S
    nblk = S - 1
    L = 16
    KPB = S // L

    nfull = nblk // _G
    rem = nblk - nfull * _G
    W = ((_G - 1) * (S + 1) + 1 + S + (nfull - 1) * _G + S - 1) // S * S
    W = 640 if (S == 128 and _G == 4) else W
    t_d0 = nfull * _G
    t_src0 = (S + 1) * t_d0 + 1
    t_A = t_src0 - t_src0 % S
    t_offs = [(S + 1) * (t_d0 + j) + 1 - t_A for j in range(rem)]
    t_W = ((t_offs[-1] + S + S - 1) // S) * S
    assert t_A + t_W <= S2 and t_W <= W

    ntrip = nfull // _NBUF
    nstat = nfull - ntrip * _NBUF

    mesh = plsc.VectorSubcoreMesh(core_axis_name="c", subcore_axis_name="s")

    @functools.partial(
        pl.kernel,
        mesh=mesh,
        out_type=jax.ShapeDtypeStruct((B, D, out_rows), arr.dtype),
        scratch_types=(
            [pltpu.VMEM((_NBUF, D, W), arr.dtype)]
            + [pltpu.SemaphoreType.DMA] * (2 * _NBUF)
        ),
        compiler_params=pltpu.CompilerParams(needs_layout_passes=False),
    )
    def copy_offdiag(arr_hbm, out_hbm, buf, *sems):
        sem_g = sems[:_NBUF]
        sem_s = sems[_NBUF:]
        wid = lax.axis_index("s") * 2 + lax.axis_index("c")
        b = wid
        iota = lax.iota(jnp.int32, L)

        def gather(ci, p):
            A = _G * S * ci
            pltpu.async_copy(
                arr_hbm.at[b, :, pl.ds(A, W)],
                buf.at[p],
                sem_g[p],
            )

        def wait_gather(ci, p):
            pltpu.make_async_copy(
                arr_hbm.at[b, :, pl.ds(_G * S * ci, W)],
                buf.at[p],
                sem_g[p],
            ).wait()

        def compact(ci, p):
            base = (_G * ci + 1)
            slot = buf.at[p]

            def row_body(r, carry):
                rvec = jnp.zeros((L,), jnp.int32) + r
                for j in range(_G):
                    for k in range(KPB):
                        src = iota + (base + (S + 1) * j + L * k)
                        dst = iota + (S * j + L * k)
                        v = plsc.load_gather(slot, [rvec, src])
                        plsc.store_scatter(slot, [rvec, dst], v)
                return carry

            lax.fori_loop(0, D, row_body, None)

        def scatter(ci, p):
            pltpu.async_copy(
                buf.at[p, :, pl.ds(0, S * _G)],
                out_hbm.at[b, :, pl.ds(_G * S * ci, S * _G)],
                sem_s[p],
            )

        def wait_scatter(ci, p):
            pltpu.make_async_copy(
                buf.at[p, :, pl.ds(0, S * _G)],
                out_hbm.at[b, :, pl.ds(_G * S * ci, S * _G)],
                sem_s[p],
            ).wait()

        for p in range(_NBUF):
            gather(p, p)

        def trip_body(t, carry):
            for par in range(_NBUF):
                ci = _NBUF * t + par
                wait_gather(ci, par)
                compact(ci, par)
                scatter(ci, par)
                nc = ci + _NBUF

                @pl.when(nc <= nfull - 1)
                def _():
                    wait_scatter(ci, par)
                    gather(nc, par)

            return carry

        lax.fori_loop(0, ntrip, trip_body, None)

        for ci in range(ntrip * _NBUF, nfull):
            par = ci % _NBUF
            wait_gather(ci, par)
            compact(ci, par)
            scatter(ci, par)

        t_par = nfull % _NBUF
        t_prev = t_par + ((nfull - 1 - t_par) // _NBUF) * _NBUF
        wait_scatter(t_prev, t_par)
        pltpu.async_copy(
            arr_hbm.at[b, :, pl.ds(t_A, t_W)],
            buf.at[t_par, :, pl.ds(0, t_W)],
            sem_g[t_par],
        )
        pltpu.make_async_copy(
            arr_hbm.at[b, :, pl.ds(t_A, t_W)],
            buf.at[t_par, :, pl.ds(0, t_W)],
            sem_g[t_par],
        ).wait()

        def t_row_body(r, carry):
            rvec = jnp.zeros((L,), jnp.int32) + r
            for j in range(rem):
                for k in range(KPB):
                    src = iota + (t_offs[j] + L * k)
                    dst = iota + (S * j + L * k)
                    v = plsc.load_gather(buf.at[t_par], [rvec, src])
                    plsc.store_scatter(buf.at[t_par], [rvec, dst], v)
            return carry

        lax.fori_loop(0, D, t_row_body, None)
        pltpu.async_copy(
            buf.at[t_par, :, pl.ds(0, S * rem)],
            out_hbm.at[b, :, pl.ds(S * t_d0, S * rem)],
            sem_s[t_par],
        )

        for ci in range(nfull - _NBUF, nfull):
            par = ci % _NBUF
            if par == t_par:
                continue
            wait_scatter(ci, par)
        pltpu.make_async_copy(
            buf.at[t_par, :, pl.ds(0, S * rem)],
            out_hbm.at[b, :, pl.ds(S * t_d0, S * rem)],
            sem_s[t_par],
        ).wait()

    arr_t = jnp.transpose(arr, (0, 2, 1))
    out_t = copy_offdiag(arr_t)
    return jnp.transpose(out_t, (0, 2, 1))
